# baseline (speedup 1.0000x reference)
import functools

import jax
import jax.numpy as jnp
from jax.experimental import pallas as pl
from jax.experimental.pallas import tpu as pltpu

_BINS = 10
_C = 80
_CPAD = 128


def _ghmc_kernel(pred_ref, code_ref, out_ref, acc_ref, *, num_steps):
    i = pl.program_id(0)

    @pl.when(i == 0)
    def _init():
        acc_ref[...] = jnp.zeros_like(acc_ref)

    x = pred_ref[...]
    code = code_ref[...]

    rows = x.shape[0]
    code_i = code.astype(jnp.int32)
    valid_b = code_i >= 2
    t_pos = (code_i & 1) == 1
    valid = jnp.where(valid_b, 1.0, 0.0)
    t = jnp.where(t_pos, 1.0, 0.0)
    sig = 1.0 / (1.0 + jnp.exp(-x))
    g10 = jnp.abs(sig - t) * 10.0

    m = jnp.max(x, axis=-1, keepdims=True)
    lse = jnp.log(jnp.sum(jnp.exp(x - m), axis=-1, keepdims=True))
    lsm = x - m - lse
    c = jnp.where(valid_b & t_pos, lsm, 0.0)

    for k in range(_BINS):
        if k == 0:
            vk, ck = valid, c
        else:
            cb = g10 >= float(k)
            vk = jnp.where(cb, valid, 0.0)
            ck = jnp.where(cb, c, 0.0)
        acc_ref[k] += jnp.sum(vk.reshape(rows // 8, 8, _CPAD), axis=0)
        acc_ref[_BINS + k] += jnp.sum(ck.reshape(rows // 8, 8, _CPAD), axis=0)

    @pl.when(i == num_steps - 1)
    def _epilogue():
        nb = jnp.float32(0.0)
        total = jnp.float32(0.0)
        for b in range(_BINS):
            cnt_lane = acc_ref[b]
            s_lane = acc_ref[_BINS + b]
            if b < _BINS - 1:
                cnt_lane = cnt_lane - acc_ref[b + 1]
                s_lane = s_lane - acc_ref[_BINS + b + 1]
            cnt = jnp.sum(cnt_lane)
            s = jnp.sum(s_lane)
            nb += (cnt > 0.0).astype(jnp.float32)
            total += s / jnp.maximum(cnt, 1.0)
        out_ref[0, 0] = -total / jnp.maximum(nb, 1.0)


@jax.jit
def kernel(pred, target, label_weight):
    n_rows = pred.shape[0]
    block_rows = 2048
    num_steps = n_rows // block_rows

    pred_p = jnp.pad(
        pred, ((0, 0), (0, _CPAD - _C)), constant_values=-jnp.inf
    )
    code = (
        (target != 0.0).astype(jnp.int8)
        + 2 * (label_weight > 0.0).astype(jnp.int8)
    )
    code = jnp.pad(code, ((0, 0), (0, _CPAD - _C)), constant_values=0)

    out = pl.pallas_call(
        functools.partial(_ghmc_kernel, num_steps=num_steps),
        grid=(num_steps,),
        in_specs=[
            pl.BlockSpec((block_rows, _CPAD), lambda i: (i, 0)),
            pl.BlockSpec((block_rows, _CPAD), lambda i: (i, 0)),
        ],
        out_specs=pl.BlockSpec(memory_space=pltpu.SMEM),
        out_shape=jax.ShapeDtypeStruct((1, 1), jnp.float32),
        scratch_shapes=[pltpu.VMEM((2 * _BINS, 8, _CPAD), jnp.float32)],
    )(pred_p, code)
    return jnp.reshape(out, ())


# speedup vs baseline: 1.1624x; 1.1624x over previous
import functools

import jax
import jax.numpy as jnp
from jax.experimental import pallas as pl
from jax.experimental.pallas import tpu as pltpu

_BINS = 10
_C = 80


def _ghmc_kernel(pred_ref, code_ref, out_ref, acc_ref, *, num_steps):
    i = pl.program_id(0)

    @pl.when(i == 0)
    def _init():
        acc_ref[...] = jnp.zeros_like(acc_ref)

    x = pred_ref[...]
    code_i = code_ref[...].astype(jnp.int32)

    rows = x.shape[0]
    t_pos = (code_i & 1) == 1
    valid = jnp.where(code_i >= 2, 1.0, 0.0)
    sig = 1.0 / (1.0 + jnp.exp(-x))
    g10 = jnp.where(t_pos, 1.0 - sig, sig) * 10.0

    m = jnp.max(x, axis=-1, keepdims=True)
    lse = jnp.log(jnp.sum(jnp.exp(x - m), axis=-1, keepdims=True))
    lsm = x - m - lse
    vt = jnp.where(t_pos, valid, 0.0)
    c = vt * lsm

    e0 = jnp.where(
        jax.lax.broadcasted_iota(jnp.int32, (8, rows), 0) == 0, 1.0, 0.0
    )

    for k in range(_BINS):
        if k == 0:
            vk, ck = valid, c
        else:
            cb = g10 >= float(k)
            vk = jnp.where(cb, valid, 0.0)
            ck = jnp.where(cb, c, 0.0)
        acc_ref[k] += jnp.dot(e0, vk, preferred_element_type=jnp.float32)
        acc_ref[_BINS + k] += jnp.dot(e0, ck, preferred_element_type=jnp.float32)

    @pl.when(i == num_steps - 1)
    def _epilogue():
        nb = jnp.float32(0.0)
        total = jnp.float32(0.0)
        for b in range(_BINS):
            cnt_lane = acc_ref[b]
            s_lane = acc_ref[_BINS + b]
            if b < _BINS - 1:
                cnt_lane = cnt_lane - acc_ref[b + 1]
                s_lane = s_lane - acc_ref[_BINS + b + 1]
            cnt = jnp.sum(cnt_lane)
            s = jnp.sum(s_lane)
            nb += (cnt > 0.0).astype(jnp.float32)
            total += s / jnp.maximum(cnt, 1.0)
        out_ref[0, 0] = -total / jnp.maximum(nb, 1.0)


@jax.jit
def kernel(pred, target, label_weight):
    n_rows = pred.shape[0]
    block_rows = 2048
    num_steps = n_rows // block_rows

    code = (
        (target != 0.0).astype(jnp.int8)
        + 2 * (label_weight > 0.0).astype(jnp.int8)
    )

    out = pl.pallas_call(
        functools.partial(_ghmc_kernel, num_steps=num_steps),
        grid=(num_steps,),
        in_specs=[
            pl.BlockSpec((block_rows, _C), lambda i: (i, 0)),
            pl.BlockSpec((block_rows, _C), lambda i: (i, 0)),
        ],
        out_specs=pl.BlockSpec(memory_space=pltpu.SMEM),
        out_shape=jax.ShapeDtypeStruct((1, 1), jnp.float32),
        scratch_shapes=[pltpu.VMEM((2 * _BINS, 8, _C), jnp.float32)],
    )(pred, code)
    return jnp.reshape(out, ())


# speedup vs baseline: 1.2628x; 1.0864x over previous
import functools

import jax
import jax.numpy as jnp
from jax.experimental import pallas as pl
from jax.experimental.pallas import tpu as pltpu

_BINS = 10
_C = 80


def _ghmc_kernel(pred_ref, tgt_ref, lw_ref, out_ref, acc_ref, *, num_steps):
    i = pl.program_id(0)

    @pl.when(i == 0)
    def _init():
        acc_ref[...] = jnp.zeros_like(acc_ref)

    x = pred_ref[...]
    t = tgt_ref[...]
    lw = lw_ref[...]

    rows = x.shape[0]
    t_pos = t > 0.0
    valid = jnp.where(lw > 0.0, 1.0, 0.0)
    sig = 1.0 / (1.0 + jnp.exp(-x))
    g10 = jnp.where(t_pos, 1.0 - sig, sig) * 10.0

    m = jnp.max(x, axis=-1, keepdims=True)
    lse = jnp.log(jnp.sum(jnp.exp(x - m), axis=-1, keepdims=True))
    lsm = x - m - lse
    vt = jnp.where(t_pos, valid, 0.0)
    c = vt * lsm

    e0 = jnp.where(
        jax.lax.broadcasted_iota(jnp.int32, (8, rows), 0) == 0, 1.0, 0.0
    )

    for k in range(_BINS):
        if k == 0:
            vk, ck = valid, c
        else:
            cb = g10 >= float(k)
            vk = jnp.where(cb, valid, 0.0)
            ck = jnp.where(cb, c, 0.0)
        acc_ref[k] += jnp.dot(e0, vk, preferred_element_type=jnp.float32)
        acc_ref[_BINS + k] += jnp.dot(e0, ck, preferred_element_type=jnp.float32)

    @pl.when(i == num_steps - 1)
    def _epilogue():
        nb = jnp.float32(0.0)
        total = jnp.float32(0.0)
        for b in range(_BINS):
            cnt_lane = acc_ref[b]
            s_lane = acc_ref[_BINS + b]
            if b < _BINS - 1:
                cnt_lane = cnt_lane - acc_ref[b + 1]
                s_lane = s_lane - acc_ref[_BINS + b + 1]
            cnt = jnp.sum(cnt_lane)
            s = jnp.sum(s_lane)
            nb += (cnt > 0.0).astype(jnp.float32)
            total += s / jnp.maximum(cnt, 1.0)
        out_ref[0, 0] = -total / jnp.maximum(nb, 1.0)


@jax.jit
def kernel(pred, target, label_weight):
    n_rows = pred.shape[0]
    block_rows = 2048
    num_steps = n_rows // block_rows

    out = pl.pallas_call(
        functools.partial(_ghmc_kernel, num_steps=num_steps),
        grid=(num_steps,),
        in_specs=[
            pl.BlockSpec((block_rows, _C), lambda i: (i, 0)),
            pl.BlockSpec((block_rows, _C), lambda i: (i, 0)),
            pl.BlockSpec((block_rows, _C), lambda i: (i, 0)),
        ],
        out_specs=pl.BlockSpec(memory_space=pltpu.SMEM),
        out_shape=jax.ShapeDtypeStruct((1, 1), jnp.float32),
        scratch_shapes=[pltpu.VMEM((2 * _BINS, 8, _C), jnp.float32)],
    )(pred, target, label_weight)
    return jnp.reshape(out, ())


# speedup vs baseline: 1.2863x; 1.0186x over previous
import functools

import jax
import jax.numpy as jnp
from jax.experimental import pallas as pl
from jax.experimental.pallas import tpu as pltpu

_BINS = 10
_C = 80


def _ghmc_kernel(pred_ref, tgt_ref, lw_ref, out_ref, acc_ref, *, num_steps):
    i = pl.program_id(0)

    @pl.when(i == 0)
    def _init():
        acc_ref[...] = jnp.zeros_like(acc_ref)

    x = pred_ref[...]
    t = tgt_ref[...]
    lw = lw_ref[...]

    rows = x.shape[0]
    t_pos = t > 0.0
    valid = jnp.where(lw > 0.0, 1.0, 0.0)
    sig = 1.0 / (1.0 + jnp.exp(-x))
    g10 = jnp.where(t_pos, 1.0 - sig, sig) * 10.0

    m = jnp.max(x, axis=-1, keepdims=True)
    lse = jnp.log(jnp.sum(jnp.exp(x - m), axis=-1, keepdims=True))
    lsm = x - m - lse
    vt = jnp.where(t_pos, valid, 0.0)
    c = vt * lsm

    e0 = jnp.where(
        jax.lax.broadcasted_iota(jnp.int32, (8, rows), 0) == 0, 1.0, 0.0
    )

    for k in range(_BINS):
        if k == 0:
            vk, ck = valid, c
        else:
            cb = g10 >= float(k)
            vk = jnp.where(cb, valid, 0.0)
            ck = jnp.where(cb, c, 0.0)
        acc_ref[k] += jnp.dot(e0, vk, preferred_element_type=jnp.float32)
        acc_ref[_BINS + k] += jnp.dot(e0, ck, preferred_element_type=jnp.float32)

    @pl.when(i == num_steps - 1)
    def _epilogue():
        nb = jnp.float32(0.0)
        total = jnp.float32(0.0)
        for b in range(_BINS):
            cnt_lane = acc_ref[b]
            s_lane = acc_ref[_BINS + b]
            if b < _BINS - 1:
                cnt_lane = cnt_lane - acc_ref[b + 1]
                s_lane = s_lane - acc_ref[_BINS + b + 1]
            cnt = jnp.sum(cnt_lane)
            s = jnp.sum(s_lane)
            nb += (cnt > 0.0).astype(jnp.float32)
            total += s / jnp.maximum(cnt, 1.0)
        out_ref[0, 0] = -total / jnp.maximum(nb, 1.0)


@jax.jit
def kernel(pred, target, label_weight):
    n_rows = pred.shape[0]
    block_rows = 4096
    num_steps = n_rows // block_rows

    out = pl.pallas_call(
        functools.partial(_ghmc_kernel, num_steps=num_steps),
        grid=(num_steps,),
        in_specs=[
            pl.BlockSpec((block_rows, _C), lambda i: (i, 0)),
            pl.BlockSpec((block_rows, _C), lambda i: (i, 0)),
            pl.BlockSpec((block_rows, _C), lambda i: (i, 0)),
        ],
        out_specs=pl.BlockSpec(memory_space=pltpu.SMEM),
        out_shape=jax.ShapeDtypeStruct((1, 1), jnp.float32),
        scratch_shapes=[pltpu.VMEM((2 * _BINS, 8, _C), jnp.float32)],
    )(pred, target, label_weight)
    return jnp.reshape(out, ())


# speedup vs baseline: 1.2977x; 1.0088x over previous
import functools

import jax
import jax.numpy as jnp
from jax.experimental import pallas as pl
from jax.experimental.pallas import tpu as pltpu

_BINS = 10
_C = 80


def _ghmc_kernel(pred_ref, tgt_ref, lw_ref, out_ref, acc_ref, *, num_steps):
    i = pl.program_id(0)

    @pl.when(i == 0)
    def _init():
        acc_ref[...] = jnp.zeros_like(acc_ref)

    x = pred_ref[...]
    t = tgt_ref[...]
    lw = lw_ref[...]

    rows = x.shape[0]
    t_pos = t > 0.0
    valid = jnp.where(lw > 0.0, 1.0, 0.0)
    sig = 1.0 / (1.0 + jnp.exp(-x))
    g10 = jnp.where(t_pos, 1.0 - sig, sig) * 10.0

    m = jnp.max(x, axis=-1, keepdims=True)
    lse = jnp.log(jnp.sum(jnp.exp(x - m), axis=-1, keepdims=True))
    lsm = x - m - lse
    vt = jnp.where(t_pos, valid, 0.0)
    c = vt * lsm

    e0 = jnp.where(
        jax.lax.broadcasted_iota(jnp.int32, (8, rows), 0) == 0, 1.0, 0.0
    )

    for k in range(_BINS):
        if k == 0:
            vk, ck = valid, c
        else:
            cb = g10 >= float(k)
            vk = jnp.where(cb, valid, 0.0)
            ck = jnp.where(cb, c, 0.0)
        acc_ref[k] += jnp.dot(e0, vk, preferred_element_type=jnp.float32)
        acc_ref[_BINS + k] += jnp.dot(e0, ck, preferred_element_type=jnp.float32)

    @pl.when(i == num_steps - 1)
    def _epilogue():
        nb = jnp.float32(0.0)
        total = jnp.float32(0.0)
        for b in range(_BINS):
            cnt_lane = acc_ref[b]
            s_lane = acc_ref[_BINS + b]
            if b < _BINS - 1:
                cnt_lane = cnt_lane - acc_ref[b + 1]
                s_lane = s_lane - acc_ref[_BINS + b + 1]
            cnt = jnp.sum(cnt_lane)
            s = jnp.sum(s_lane)
            nb += (cnt > 0.0).astype(jnp.float32)
            total += s / jnp.maximum(cnt, 1.0)
        out_ref[0, 0] = -total / jnp.maximum(nb, 1.0)


@jax.jit
def kernel(pred, target, label_weight):
    n_rows = pred.shape[0]
    block_rows = 8192
    num_steps = n_rows // block_rows

    out = pl.pallas_call(
        functools.partial(_ghmc_kernel, num_steps=num_steps),
        grid=(num_steps,),
        in_specs=[
            pl.BlockSpec((block_rows, _C), lambda i: (i, 0)),
            pl.BlockSpec((block_rows, _C), lambda i: (i, 0)),
            pl.BlockSpec((block_rows, _C), lambda i: (i, 0)),
        ],
        out_specs=pl.BlockSpec(memory_space=pltpu.SMEM),
        out_shape=jax.ShapeDtypeStruct((1, 1), jnp.float32),
        scratch_shapes=[pltpu.VMEM((2 * _BINS, 8, _C), jnp.float32)],
    )(pred, target, label_weight)
    return jnp.reshape(out, ())
